# Initial kernel scaffold; baseline (speedup 1.0000x reference)
import jax
import jax.numpy as jnp
from jax.experimental import pallas as pl


def kernel(pos, table):
    raise NotImplementedError("write your pallas kernel here")


# speedup vs baseline: 4.1639x; 4.1639x over previous
import functools
import math

import jax
import jax.numpy as jnp
from jax import lax
from jax.experimental import pallas as pl
from jax.experimental.pallas import tpu as pltpu
from jax.experimental.pallas import tpu_sc as plsc

MAXLEN = 200
EMB = 128
NC, NS = 2, 16
NW = NC * NS
N = 1024 * MAXLEN
PER_W = N // NW
CH = 128
NCH = PER_W // CH


def _rel_body(o_ref):
    pi = lax.broadcasted_iota(jnp.int32, (MAXLEN, EMB), 0)
    ci = lax.broadcasted_iota(jnp.int32, (MAXLEN, EMB), 1)
    p = pi.astype(jnp.float32)
    c = ci.astype(jnp.float32)
    div = jnp.exp(c * (math.log(10000.0) / (EMB // 2)))
    arg = p / div
    o_ref[...] = jnp.where(ci < EMB // 2, jnp.sin(arg), jnp.cos(arg))


@functools.cache
def _make_sc_gather():
    mesh = plsc.VectorSubcoreMesh(
        core_axis_name="c", subcore_axis_name="s",
        num_cores=NC, num_subcores=NS)

    @functools.partial(
        pl.kernel,
        out_type=(
            jax.ShapeDtypeStruct((N, EMB), jnp.float32),
            jax.ShapeDtypeStruct((N, EMB), jnp.float32),
        ),
        mesh=mesh,
        scratch_types=[
            pltpu.VMEM((NCH, CH), jnp.int32),
            pltpu.VMEM((CH, EMB), jnp.float32),
            pltpu.VMEM((CH, EMB), jnp.float32),
            pltpu.SemaphoreType.DMA,
            pltpu.SemaphoreType.DMA,
        ],
    )
    def sc_gather(table_hbm, rel_hbm, idx_hbm, out_a, out_r,
                  idx_v, rows_a, rows_r, sem_a, sem_r):
        wid = lax.axis_index("s") * NC + lax.axis_index("c")
        base = wid * PER_W
        pltpu.sync_copy(idx_hbm.at[wid], idx_v)

        @pl.loop(0, NCH)
        def _chunk(j):
            ga = pltpu.async_copy(table_hbm.at[idx_v.at[j]], rows_a, sem_a)
            gr = pltpu.async_copy(rel_hbm.at[idx_v.at[j]], rows_r, sem_r)
            ga.wait()
            pltpu.sync_copy(rows_a, out_a.at[pl.ds(base + j * CH, CH)])
            gr.wait()
            pltpu.sync_copy(rows_r, out_r.at[pl.ds(base + j * CH, CH)])

    return sc_gather


def kernel(pos, table):
    rel_tab = pl.pallas_call(
        _rel_body,
        out_shape=jax.ShapeDtypeStruct((MAXLEN, EMB), jnp.float32),
    )()
    idx = pos.reshape(NW, NCH, CH)
    out_a, out_r = _make_sc_gather()(table, rel_tab, idx)
    b, l = pos.shape
    return out_a.reshape(b, l, EMB), out_r.reshape(b, l, EMB)


# speedup vs baseline: 4.2466x; 1.0199x over previous
import functools
import math

import jax
import jax.numpy as jnp
from jax import lax
from jax.experimental import pallas as pl
from jax.experimental.pallas import tpu as pltpu
from jax.experimental.pallas import tpu_sc as plsc

MAXLEN = 200
EMB = 128
NC, NS = 2, 16
NW = NC * NS
N = 1024 * MAXLEN
PER_W = N // NW
CH = 64
NCH = PER_W // CH
NBUF = 4


def _rel_body(o_ref):
    pi = lax.broadcasted_iota(jnp.int32, (MAXLEN, EMB), 0)
    ci = lax.broadcasted_iota(jnp.int32, (MAXLEN, EMB), 1)
    p = pi.astype(jnp.float32)
    c = ci.astype(jnp.float32)
    div = jnp.exp(c * (math.log(10000.0) / (EMB // 2)))
    arg = p / div
    o_ref[...] = jnp.where(ci < EMB // 2, jnp.sin(arg), jnp.cos(arg))


@functools.cache
def _make_sc_gather():
    mesh = plsc.VectorSubcoreMesh(
        core_axis_name="c", subcore_axis_name="s",
        num_cores=NC, num_subcores=NS)

    row_buf = pltpu.VMEM((CH, EMB), jnp.float32)
    dma = pltpu.SemaphoreType.DMA

    @functools.partial(
        pl.kernel,
        out_type=(
            jax.ShapeDtypeStruct((N, EMB), jnp.float32),
            jax.ShapeDtypeStruct((N, EMB), jnp.float32),
        ),
        mesh=mesh,
        scratch_types=(
            [pltpu.VMEM((NCH, CH), jnp.int32)]
            + [row_buf] * (2 * NBUF)
            + [dma] * (4 * NBUF)
        ),
    )
    def sc_gather(table_hbm, rel_hbm, idx_hbm, out_a, out_r,
                  idx_v,
                  ba0, ba1, ba2, ba3, br0, br1, br2, br3,
                  gsa0, gsa1, gsa2, gsa3, gsr0, gsr1, gsr2, gsr3,
                  wsa0, wsa1, wsa2, wsa3, wsr0, wsr1, wsr2, wsr3):
        ba = (ba0, ba1, ba2, ba3)
        br = (br0, br1, br2, br3)
        gsa = (gsa0, gsa1, gsa2, gsa3)
        gsr = (gsr0, gsr1, gsr2, gsr3)
        wsa = (wsa0, wsa1, wsa2, wsa3)
        wsr = (wsr0, wsr1, wsr2, wsr3)

        wid = lax.axis_index("s") * NC + lax.axis_index("c")
        base = wid * PER_W
        pltpu.sync_copy(idx_hbm.at[wid], idx_v)

        def gstart(j, s):
            pltpu.async_copy(table_hbm.at[idx_v.at[j]], ba[s], gsa[s])
            pltpu.async_copy(rel_hbm.at[idx_v.at[j]], br[s], gsr[s])

        def gwait(s):
            pltpu.make_async_copy(table_hbm.at[idx_v.at[0]], ba[s], gsa[s]).wait()
            pltpu.make_async_copy(rel_hbm.at[idx_v.at[0]], br[s], gsr[s]).wait()

        def wstart(j, s):
            dst = pl.ds(base + j * CH, CH)
            pltpu.async_copy(ba[s], out_a.at[dst], wsa[s])
            pltpu.async_copy(br[s], out_r.at[dst], wsr[s])

        def wwait(s):
            dst = pl.ds(base, CH)
            pltpu.make_async_copy(ba[s], out_a.at[dst], wsa[s]).wait()
            pltpu.make_async_copy(br[s], out_r.at[dst], wsr[s]).wait()

        gstart(0, 0)
        gstart(1, 1)

        @pl.loop(0, NCH, step=NBUF)
        def _outer(i0):
            for b in range(NBUF):
                i = i0 + b
                s = b
                s2 = (b + 2) % NBUF
                gwait(s)
                wstart(i, s)

                @pl.when(i >= 2)
                def _():
                    wwait(s2)

                @pl.when(i + 2 < NCH)
                def _():
                    gstart(i + 2, s2)

        wwait(2)
        wwait(3)

    return sc_gather


def kernel(pos, table):
    rel_tab = pl.pallas_call(
        _rel_body,
        out_shape=jax.ShapeDtypeStruct((MAXLEN, EMB), jnp.float32),
    )()
    idx = pos.reshape(NW, NCH, CH)
    out_a, out_r = _make_sc_gather()(table, rel_tab, idx)
    b, l = pos.shape
    return out_a.reshape(b, l, EMB), out_r.reshape(b, l, EMB)
